# Initial kernel scaffold; baseline (speedup 1.0000x reference)
import jax
import jax.numpy as jnp
from jax.experimental import pallas as pl


def kernel(x, Wg, We, w1, b1, w2, b2):
    raise NotImplementedError("write your pallas kernel here")


# speedup vs baseline: 3.6696x; 3.6696x over previous
import functools

import jax
import jax.numpy as jnp
import numpy as np
from jax import lax
from jax.experimental import pallas as pl
from jax.experimental.pallas import tpu as pltpu

D = 768
DFF = 3072
G = 4
EG = 2
E = G * EG
TOPK = 2
N = 2048

BT = 256

_SWAP = np.zeros((E, E), np.float32)
for _i in range(E):
    _SWAP[_i ^ 1, _i] = 1.0
_EXPAND = np.zeros((G, E), np.float32)
for _g in range(G):
    _EXPAND[_g, 2 * _g] = 1.0
    _EXPAND[_g, 2 * _g + 1] = 1.0


def _precise_div(a, b):
    r = 1.0 / b
    r = r * (2.0 - b * r)
    r = r * (2.0 - b * r)
    q = a * r
    q = q + (a - q * b) * r
    return q


def _router_kernel(x_ref, wg_ref, we_ref, swap_ref, expand_ref, out_ref):
    x = x_ref[...]
    gl = lax.dot_general(x, wg_ref[...], (((1,), (1,)), ((), ())),
                         preferred_element_type=jnp.float32)
    gmax = jnp.max(gl, axis=-1, keepdims=True)
    gexp = jnp.exp(gl - gmax)
    gp = _precise_div(gexp, jnp.sum(gexp, axis=-1, keepdims=True))
    el = lax.dot_general(x, we_ref[...], (((1,), (1,)), ((), ())),
                         preferred_element_type=jnp.float32)
    swap = swap_ref[...]
    partner = lax.dot_general(el, swap, (((1,), (0,)), ((), ())),
                              preferred_element_type=jnp.float32,
                              precision=lax.Precision.HIGHEST)
    emax = jnp.maximum(el, partner)
    eexp = jnp.exp(el - emax)
    pexp = lax.dot_general(eexp, swap, (((1,), (0,)), ((), ())),
                           preferred_element_type=jnp.float32,
                           precision=lax.Precision.HIGHEST)
    ep = _precise_div(eexp, eexp + pexp)
    gp8 = lax.dot_general(gp, expand_ref[...], (((1,), (0,)), ((), ())),
                          preferred_element_type=jnp.float32,
                         precision=lax.Precision.HIGHEST)
    valid = (jnp.where(gp8 >= (1.0 / G), 1.0, 0.0)
             * jnp.where(ep >= (1.0 / EG), 1.0, 0.0))
    fp = gp8 * ep
    nsel = jnp.sum(valid, axis=-1, keepdims=True)
    lanes = lax.broadcasted_iota(jnp.int32, fp.shape, 1)
    m1 = jnp.max(fp, axis=-1, keepdims=True)
    i1 = jnp.min(jnp.where(fp == m1, lanes, E), axis=-1, keepdims=True)
    fp2 = jnp.where(lanes == i1, -1.0, fp)
    m2 = jnp.max(fp2, axis=-1, keepdims=True)
    i2 = jnp.min(jnp.where(fp2 == m2, lanes, E), axis=-1, keepdims=True)
    topk_mask = (jnp.where(lanes == i1, 1.0, 0.0)
                 + jnp.where(lanes == i2, 1.0, 0.0))
    final_mask = jnp.where(nsel < TOPK, topk_mask, valid)
    sel_w = fp * final_mask
    wsum = jnp.maximum(jnp.sum(sel_w, axis=-1, keepdims=True), 1e-9)
    out_ref[...] = _precise_div(sel_w, wsum)


def _ffn_kernel(x_ref, w1_ref, b1_ref, w2_ref, b2_ref, wts_ref, out_ref):
    e = pl.program_id(0)
    j = pl.program_id(1)
    xb = x_ref[...]
    w1e = w1_ref[0]
    h = lax.dot_general(xb, w1e, (((1,), (1,)), ((), ())),
                        preferred_element_type=jnp.float32)
    h = h + b1_ref[pl.ds(e, 1), :]
    h = 0.5 * h * (1.0 + lax.erf(h * np.float32(1.0 / np.sqrt(2.0))))
    w2e = w2_ref[0]
    o = lax.dot_general(h, w2e, (((1,), (1,)), ((), ())),
                        preferred_element_type=jnp.float32)
    o = o + b2_ref[pl.ds(e, 1), :]
    lanes = lax.broadcasted_iota(jnp.int32, wts_ref.shape, 1)
    wcol = jnp.sum(jnp.where(lanes == e, wts_ref[...], 0.0),
                   axis=-1, keepdims=True)
    contrib = o * wcol

    @pl.when(e == 0)
    def _init():
        out_ref[pl.ds(j * BT, BT), :] = contrib

    @pl.when(e != 0)
    def _acc():
        out_ref[pl.ds(j * BT, BT), :] += contrib


@jax.jit
def kernel(x, Wg, We, w1, b1, w2, b2):
    weights = pl.pallas_call(
        _router_kernel,
        out_shape=jax.ShapeDtypeStruct((N, E), jnp.float32),
    )(x, Wg, We, jnp.asarray(_SWAP), jnp.asarray(_EXPAND))

    out = pl.pallas_call(
        _ffn_kernel,
        grid=(E, N // BT),
        in_specs=[
            pl.BlockSpec((BT, D), lambda e, j: (j, 0)),
            pl.BlockSpec((1, DFF, D), lambda e, j: (e, 0, 0)),
            pl.BlockSpec((E, DFF), lambda e, j: (0, 0)),
            pl.BlockSpec((1, D, DFF), lambda e, j: (e, 0, 0)),
            pl.BlockSpec((E, D), lambda e, j: (0, 0)),
            pl.BlockSpec((BT, E), lambda e, j: (j, 0)),
        ],
        out_specs=pl.BlockSpec((N, D), lambda e, j: (0, 0)),
        out_shape=jax.ShapeDtypeStruct((N, D), jnp.float32),
    )(x, w1, b1, w2, b2, weights)
    return out


# speedup vs baseline: 4.3027x; 1.1725x over previous
import functools

import jax
import jax.numpy as jnp
import numpy as np
from jax import lax
from jax.experimental import pallas as pl
from jax.experimental.pallas import tpu as pltpu

D = 768
DFF = 3072
G = 4
EG = 2
E = G * EG
TOPK = 2
N = 2048

BT = 512

_SWAP = np.zeros((E, E), np.float32)
for _i in range(E):
    _SWAP[_i ^ 1, _i] = 1.0
_EXPAND = np.zeros((G, E), np.float32)
for _g in range(G):
    _EXPAND[_g, 2 * _g] = 1.0
    _EXPAND[_g, 2 * _g + 1] = 1.0


def _precise_div(a, b):
    r = 1.0 / b
    r = r * (2.0 - b * r)
    r = r * (2.0 - b * r)
    q = a * r
    q = q + (a - q * b) * r
    return q


def _router_kernel(x_ref, wg_ref, we_ref, swap_ref, expand_ref, out_ref):
    x = x_ref[...]
    gl = lax.dot_general(x, wg_ref[...], (((1,), (1,)), ((), ())),
                         preferred_element_type=jnp.float32)
    gmax = jnp.max(gl, axis=-1, keepdims=True)
    gexp = jnp.exp(gl - gmax)
    gp = _precise_div(gexp, jnp.sum(gexp, axis=-1, keepdims=True))
    el = lax.dot_general(x, we_ref[...], (((1,), (1,)), ((), ())),
                         preferred_element_type=jnp.float32)
    swap = swap_ref[...]
    partner = lax.dot_general(el, swap, (((1,), (0,)), ((), ())),
                              preferred_element_type=jnp.float32,
                              precision=lax.Precision.HIGHEST)
    emax = jnp.maximum(el, partner)
    eexp = jnp.exp(el - emax)
    pexp = lax.dot_general(eexp, swap, (((1,), (0,)), ((), ())),
                           preferred_element_type=jnp.float32,
                           precision=lax.Precision.HIGHEST)
    ep = _precise_div(eexp, eexp + pexp)
    gp8 = lax.dot_general(gp, expand_ref[...], (((1,), (0,)), ((), ())),
                          preferred_element_type=jnp.float32,
                         precision=lax.Precision.HIGHEST)
    valid = (jnp.where(gp8 >= (1.0 / G), 1.0, 0.0)
             * jnp.where(ep >= (1.0 / EG), 1.0, 0.0))
    fp = gp8 * ep
    nsel = jnp.sum(valid, axis=-1, keepdims=True)
    lanes = lax.broadcasted_iota(jnp.int32, fp.shape, 1)
    m1 = jnp.max(fp, axis=-1, keepdims=True)
    i1 = jnp.min(jnp.where(fp == m1, lanes, E), axis=-1, keepdims=True)
    fp2 = jnp.where(lanes == i1, -1.0, fp)
    m2 = jnp.max(fp2, axis=-1, keepdims=True)
    i2 = jnp.min(jnp.where(fp2 == m2, lanes, E), axis=-1, keepdims=True)
    topk_mask = (jnp.where(lanes == i1, 1.0, 0.0)
                 + jnp.where(lanes == i2, 1.0, 0.0))
    final_mask = jnp.where(nsel < TOPK, topk_mask, valid)
    sel_w = fp * final_mask
    wsum = jnp.maximum(jnp.sum(sel_w, axis=-1, keepdims=True), 1e-9)
    out_ref[...] = _precise_div(sel_w, wsum)


def _ffn_kernel(x_ref, w1_ref, b1_ref, w2_ref, b2_ref, wts_ref, out_ref):
    e = pl.program_id(0)
    j = pl.program_id(1)
    xb = x_ref[pl.ds(j * BT, BT), :]
    w1e = w1_ref[0]
    h = lax.dot_general(xb, w1e, (((1,), (1,)), ((), ())),
                        preferred_element_type=jnp.float32)
    h = h + b1_ref[pl.ds(e, 1), :]
    h = 0.5 * h * (1.0 + lax.erf(h * np.float32(1.0 / np.sqrt(2.0))))
    w2e = w2_ref[0]
    o = lax.dot_general(h, w2e, (((1,), (1,)), ((), ())),
                        preferred_element_type=jnp.float32)
    o = o + b2_ref[pl.ds(e, 1), :]
    lanes = lax.broadcasted_iota(jnp.int32, wts_ref.shape, 1)
    wcol = jnp.sum(jnp.where(lanes == e, wts_ref[...], 0.0),
                   axis=-1, keepdims=True)
    contrib = o * wcol

    @pl.when(e == 0)
    def _init():
        out_ref[pl.ds(j * BT, BT), :] = contrib

    @pl.when(e != 0)
    def _acc():
        out_ref[pl.ds(j * BT, BT), :] += contrib


@jax.jit
def kernel(x, Wg, We, w1, b1, w2, b2):
    weights = pl.pallas_call(
        _router_kernel,
        out_shape=jax.ShapeDtypeStruct((N, E), jnp.float32),
    )(x, Wg, We, jnp.asarray(_SWAP), jnp.asarray(_EXPAND))

    out = pl.pallas_call(
        _ffn_kernel,
        grid=(E, N // BT),
        in_specs=[
            pl.BlockSpec((N, D), lambda e, j: (0, 0)),
            pl.BlockSpec((1, DFF, D), lambda e, j: (e, 0, 0)),
            pl.BlockSpec((E, DFF), lambda e, j: (0, 0)),
            pl.BlockSpec((1, D, DFF), lambda e, j: (e, 0, 0)),
            pl.BlockSpec((E, D), lambda e, j: (0, 0)),
            pl.BlockSpec((BT, E), lambda e, j: (j, 0)),
        ],
        out_specs=pl.BlockSpec((N, D), lambda e, j: (0, 0)),
        out_shape=jax.ShapeDtypeStruct((N, D), jnp.float32),
    )(x, w1, b1, w2, b2, weights)
    return out
